# baseline (speedup 1.0000x reference)
import functools

import jax
import jax.numpy as jnp
from jax import lax
from jax.experimental import pallas as pl
from jax.experimental.pallas import tpu as pltpu
from jax.experimental.pallas import tpu_sc as plsc

NC = 2
NS = 16
L = 16
NW = NC * NS

B = 16384
D = 64
BPW = B // NW
CH = 128
NCH = BPW // CH

_mesh = plsc.VectorSubcoreMesh(core_axis_name="c", subcore_axis_name="s")


@functools.partial(
    pl.kernel,
    mesh=_mesh,
    out_type=jax.ShapeDtypeStruct((B,), jnp.float32),
    compiler_params=pltpu.CompilerParams(
        needs_layout_passes=False, use_tc_tiling_on_sc=False),
    scratch_types=[
        pltpu.VMEM((NCH, CH), jnp.int32),
        pltpu.VMEM((NCH, CH), jnp.int32),
        pltpu.VMEM((BPW, D), jnp.float32),
        pltpu.VMEM((BPW, D), jnp.float32),
        pltpu.VMEM((BPW,), jnp.float32),
        pltpu.VMEM((BPW,), jnp.float32),
        pltpu.VMEM((BPW,), jnp.float32),
        pltpu.SemaphoreType.DMA,
    ],
)
def _mf_fwd(user_hbm, item_hbm, ue_hbm, ie_hbm, ub_hbm, ib_hbm, out_hbm,
            uidx, iidx, urows, irows, ubias, ibias, outv, sem):
    wid = lax.axis_index("c") * NS + lax.axis_index("s")
    base = pl.multiple_of(wid * BPW, BPW)

    crow = pl.multiple_of(wid * NCH, NCH)
    pltpu.sync_copy(user_hbm.at[pl.ds(crow, NCH)], uidx)
    pltpu.sync_copy(item_hbm.at[pl.ds(crow, NCH)], iidx)

    copies = []
    for j in range(NCH):
        dst = pl.ds(j * CH, CH)
        copies.append(pltpu.async_copy(ue_hbm.at[uidx.at[j]], urows.at[dst], sem))
        copies.append(pltpu.async_copy(ie_hbm.at[iidx.at[j]], irows.at[dst], sem))
        copies.append(pltpu.async_copy(ub_hbm.at[uidx.at[j]], ubias.at[dst], sem))
        copies.append(pltpu.async_copy(ib_hbm.at[iidx.at[j]], ibias.at[dst], sem))
    for c in copies:
        c.wait()

    lane = lax.iota(jnp.int32, L)
    last_lane = lane == (L - 1)
    G = 16

    def body(g, carry):
        gb = pl.multiple_of(g * G, G)
        for k in range(G):
            b = gb + k
            acc = urows[b, pl.ds(0, L)] * irows[b, pl.ds(0, L)]
            for c in range(1, D // L):
                acc = acc + urows[b, pl.ds(c * L, L)] * irows[b, pl.ds(c * L, L)]
            total = plsc.cumsum(acc)
            idxv = jnp.full((L,), b, jnp.int32)
            plsc.store_scatter(outv, [idxv], total, mask=last_lane)
        return carry

    lax.fori_loop(0, BPW // G, body, 0)

    for g in range(BPW // L):
        s = pl.ds(g * L, L)
        outv[s] = outv[s] + ubias[s] + ibias[s]

    pltpu.sync_copy(outv, out_hbm.at[pl.ds(base, BPW)])


def kernel(user, item, user_e, item_e, user_b, item_b):
    u2 = user.astype(jnp.int32).reshape(B // CH, CH)
    i2 = item.astype(jnp.int32).reshape(B // CH, CH)
    return _mf_fwd(u2, i2, user_e, item_e,
                   user_b.reshape(-1), item_b.reshape(-1))
